# Initial kernel scaffold; baseline (speedup 1.0000x reference)
import functools

import jax
import jax.numpy as jnp
from jax import lax
from jax.experimental import pallas as pl
from jax.experimental.pallas import tpu as pltpu
from jax.experimental.pallas import tpu_sc as plsc

VOCAB = 100000
D = 64
B = 4096
S = 200

NC = 2
NS = 16
NW = NC * NS
B_PER_W = B // NW
HALF = S // 2
NREG = D // 16


def _bow_body(inputs_hbm, table_hbm, bias_hbm, out_hbm,
              idx_v, rows_v, out_v, bias_v, sem):
    wid = lax.axis_index("s") * NC + lax.axis_index("c")
    base = wid * B_PER_W

    pltpu.sync_copy(inputs_hbm.at[pl.ds(base, B_PER_W)], idx_v)
    pltpu.sync_copy(bias_hbm, bias_v)
    bias_regs = [bias_v[pl.ds(16 * d, 16)] for d in range(NREG)]

    def row_body(r, carry):
        cp0 = pltpu.async_copy(table_hbm.at[idx_v.at[r, 0]], rows_v.at[0], sem)
        cp1 = pltpu.async_copy(table_hbm.at[idx_v.at[r, 1]], rows_v.at[1], sem)
        cp0.wait()
        cp1.wait()

        def seq_body(t, acc):
            new = list(acc)
            for j in range(2):
                for d in range(NREG):
                    new[j * NREG + d] = new[j * NREG + d] + rows_v[j, t, pl.ds(16 * d, 16)]
            return tuple(new)

        zero = jnp.zeros((16,), jnp.float32)
        acc = lax.fori_loop(0, HALF, seq_body, (zero,) * (2 * NREG))
        for d in range(NREG):
            out_v[r, pl.ds(16 * d, 16)] = acc[d] + acc[NREG + d] + bias_regs[d]
        return carry

    lax.fori_loop(0, B_PER_W, row_body, 0)
    pltpu.sync_copy(out_v, out_hbm.at[pl.ds(base, B_PER_W)])


def _bow(inputs3, table, bias):
    mesh = plsc.VectorSubcoreMesh(core_axis_name="c", subcore_axis_name="s")
    kern = functools.partial(
        pl.kernel,
        mesh=mesh,
        out_type=jax.ShapeDtypeStruct((B, D), jnp.float32),
        scratch_types=[
            pltpu.VMEM((B_PER_W, 2, HALF), jnp.int32),
            pltpu.VMEM((2, HALF, D), jnp.float32),
            pltpu.VMEM((B_PER_W, D), jnp.float32),
            pltpu.VMEM((D,), jnp.float32),
            pltpu.SemaphoreType.DMA,
        ],
    )(_bow_body)
    return kern(inputs3, table, bias)


def kernel(inputs, embed_weight, bias):
    inputs3 = inputs.astype(jnp.int32).reshape(B, 2, HALF)
    return _bow(inputs3, embed_weight, bias)


# speedup vs baseline: 9.3149x; 9.3149x over previous
import functools

import jax
import jax.numpy as jnp
from jax import lax
from jax.experimental import pallas as pl
from jax.experimental.pallas import tpu as pltpu
from jax.experimental.pallas import tpu_sc as plsc

VOCAB = 100000
D = 64
B = 4096
S = 200

NC = 2
NS = 16
NW = NC * NS
B_PER_W = B // NW
HALF = S // 2
NREG = D // 16


def _bow_body(inputs_hbm, table_hbm, bias_hbm, out_hbm,
              idx_v, rows_v, out_v, bias_v, sem):
    wid = lax.axis_index("s") * NC + lax.axis_index("c")
    base = wid * B_PER_W

    pltpu.sync_copy(inputs_hbm.at[pl.ds(base, B_PER_W)], idx_v)
    pltpu.sync_copy(bias_hbm, bias_v)
    bias_regs = [bias_v[pl.ds(16 * d, 16)] for d in range(NREG)]

    def row_body(r, carry):
        cp0 = pltpu.async_copy(table_hbm.at[idx_v.at[r, 0]], rows_v.at[0], sem)
        cp1 = pltpu.async_copy(table_hbm.at[idx_v.at[r, 1]], rows_v.at[1], sem)
        cp0.wait()
        cp1.wait()

        def seq_body(t, acc):
            new = list(acc)
            for j in range(2):
                for d in range(NREG):
                    new[j * NREG + d] = new[j * NREG + d] + rows_v[j, t, pl.ds(16 * d, 16)]
            return tuple(new)

        zero = jnp.zeros((16,), jnp.float32)
        acc = lax.fori_loop(0, HALF, seq_body, (zero,) * (2 * NREG))
        for d in range(NREG):
            out_v[r, pl.ds(16 * d, 16)] = acc[d] + acc[NREG + d] + bias_regs[d]
        return carry

    lax.fori_loop(0, B_PER_W, row_body, 0)
    pltpu.sync_copy(out_v, out_hbm.at[pl.ds(base, B_PER_W)])


def _bow(inputs3, table, bias):
    mesh = plsc.VectorSubcoreMesh(core_axis_name="c", subcore_axis_name="s")
    kern = functools.partial(
        pl.kernel,
        mesh=mesh,
        out_type=jax.ShapeDtypeStruct((B, D), jnp.float32),
        scratch_types=[
            pltpu.VMEM((B_PER_W, 2, HALF), jnp.int32),
            pltpu.VMEM((2, HALF, D), jnp.float32),
            pltpu.VMEM((B_PER_W, D), jnp.float32),
            pltpu.VMEM((D,), jnp.float32),
            pltpu.SemaphoreType.DMA,
        ],
        compiler_params=pltpu.CompilerParams(use_tc_tiling_on_sc=False),
    )(_bow_body)
    return kern(inputs3, table, bias)


def kernel(inputs, embed_weight, bias):
    inputs3 = inputs.astype(jnp.int32).reshape(B, 2, HALF)
    return _bow(inputs3, embed_weight, bias)


# speedup vs baseline: 16.7706x; 1.8004x over previous
import functools

import jax
import jax.numpy as jnp
from jax import lax
from jax.experimental import pallas as pl
from jax.experimental.pallas import tpu as pltpu
from jax.experimental.pallas import tpu_sc as plsc

VOCAB = 100000
D = 64
B = 4096
S = 200

NC = 2
NS = 16
NW = NC * NS
B_PER_W = B // NW
HALF = S // 2
NREG = D // 16


NBUF = 4


def _bow_body(inputs_hbm, table_hbm, bias_hbm, out_hbm,
              idx_v, rows_v, out_v, bias_v, sem0, sem1, sem2, sem3):
    sems = (sem0, sem1, sem2, sem3)
    wid = lax.axis_index("s") * NC + lax.axis_index("c")
    base = wid * B_PER_W

    pltpu.sync_copy(inputs_hbm.at[pl.ds(base, B_PER_W)], idx_v)
    pltpu.sync_copy(bias_hbm, bias_v)
    bias_regs = [bias_v[pl.ds(16 * d, 16)] for d in range(NREG)]

    def issue(r, slot):
        for j in range(2):
            pltpu.async_copy(table_hbm.at[idx_v.at[r, j]],
                             rows_v.at[slot, j], sems[slot])

    for r in range(NBUF - 1):
        issue(r, r)

    def group_body(g, carry):
        for b in range(NBUF):
            r = g * NBUF + b
            r_next = r + NBUF - 1
            slot_next = (b + NBUF - 1) % NBUF

            @pl.when(r_next < B_PER_W)
            def _():
                issue(r_next, slot_next)

            for j in range(2):
                pltpu.make_async_copy(table_hbm.at[idx_v.at[r, j]],
                                      rows_v.at[b, j], sems[b]).wait()

            def seq_body(t, acc):
                new = list(acc)
                for j in range(2):
                    for d in range(NREG):
                        new[j * NREG + d] = (new[j * NREG + d]
                                             + rows_v[b, j, t, pl.ds(16 * d, 16)])
                return tuple(new)

            zero = jnp.zeros((16,), jnp.float32)
            acc = lax.fori_loop(0, HALF, seq_body, (zero,) * (2 * NREG))
            for d in range(NREG):
                out_v[r, pl.ds(16 * d, 16)] = acc[d] + acc[NREG + d] + bias_regs[d]
        return carry

    lax.fori_loop(0, B_PER_W // NBUF, group_body, 0)
    pltpu.sync_copy(out_v, out_hbm.at[pl.ds(base, B_PER_W)])


def _bow(inputs3, table, bias):
    mesh = plsc.VectorSubcoreMesh(core_axis_name="c", subcore_axis_name="s")
    kern = functools.partial(
        pl.kernel,
        mesh=mesh,
        out_type=jax.ShapeDtypeStruct((B, D), jnp.float32),
        scratch_types=[
            pltpu.VMEM((B_PER_W, 2, HALF), jnp.int32),
            pltpu.VMEM((NBUF, 2, HALF, D), jnp.float32),
            pltpu.VMEM((B_PER_W, D), jnp.float32),
            pltpu.VMEM((D,), jnp.float32),
            pltpu.SemaphoreType.DMA,
            pltpu.SemaphoreType.DMA,
            pltpu.SemaphoreType.DMA,
            pltpu.SemaphoreType.DMA,
        ],
        compiler_params=pltpu.CompilerParams(use_tc_tiling_on_sc=False),
    )(_bow_body)
    return kern(inputs3, table, bias)


def kernel(inputs, embed_weight, bias):
    inputs3 = inputs.astype(jnp.int32).reshape(B, 2, HALF)
    return _bow(inputs3, embed_weight, bias)
